# Initial kernel scaffold; baseline (speedup 1.0000x reference)
import jax
import jax.numpy as jnp
from jax.experimental import pallas as pl


def kernel(latents_in, emb_weights):
    raise NotImplementedError("write your pallas kernel here")


# speedup vs baseline: 4.5453x; 4.5453x over previous
import jax
import jax.numpy as jnp
from jax import lax
from jax.experimental import pallas as pl
from jax.experimental.pallas import tpu as pltpu

NM = 6
DEPTH = 6
DIM = 300
BATCH = 8192
KS = [2 ** (i + 1) for i in range(DEPTH)]
OFFS = [2 ** (i + 1) - 2 for i in range(DEPTH)]
KTOT = sum(KS)
KPAD = 128
RB = 512
NBLK = BATCH // RB
DPAD = 8


def _norm_kernel(raw_ref, en_ref):
    x = raw_ref[...]
    n = jnp.sqrt(jnp.sum(x * x, axis=1, keepdims=True))
    en_ref[...] = x / jnp.maximum(n, 1e-12)


def _main_kernel(x_ref, en_ref, routs_ref, vecs_ref, stats_ref, ec_ref, loss_ref):
    m = pl.program_id(0)
    b = pl.program_id(1)
    x = x_ref[0]
    n = jnp.sqrt(jnp.sum(x * x, axis=1, keepdims=True))
    xn = x / jnp.maximum(n, 1e-12)
    en = en_ref[...]
    s = lax.dot_general(xn, en, (((1,), (1,)), ((), ())),
                        preferred_element_type=jnp.float32,
                        precision=lax.Precision.DEFAULT)
    d = 1.0 - s
    lane = lax.broadcasted_iota(jnp.int32, (RB, KPAD), 1)

    @pl.when(b == 0)
    def _init():
        stats_ref[m, 0] = jnp.full((DPAD, KPAD), -jnp.inf, jnp.float32)
        stats_ref[m, 1] = jnp.zeros((DPAD, KPAD), jnp.float32)
        stats_ref[m, 2] = jnp.broadcast_to(s[0:1, :], (DPAD, KPAD))
        for lev in range(DEPTH):
            ec_ref[m, lev] = 0.0

    prev = None
    rout_cols = []
    lev_max = []
    lev_any = []
    ec_adds = []
    for lev in range(DEPTH):
        off = OFFS[lev]
        k = KS[lev]
        in_lev = (lane >= off) & (lane < off + k)
        if lev == 0:
            valid = in_lev
        else:
            valid = in_lev & (((lane - off) >> 1) == prev)
        dm = jnp.where(valid, d, jnp.inf)
        dmin = jnp.min(dm, axis=1, keepdims=True)
        hit = valid & (d == dmin)
        gcol = jnp.min(jnp.where(hit, lane, 2 * KPAD), axis=1, keepdims=True)
        prev = gcol - off
        rout_cols.append(prev)
        ec_adds.append(jnp.sum(1.0 - dmin))
        sm = jnp.where(valid, s, -jnp.inf)
        lev_max.append(jnp.max(sm, axis=0, keepdims=True))
        lev_any.append(jnp.max(jnp.where(valid, 1.0, 0.0), axis=0, keepdims=True))
        oh = jnp.where(gcol == lane, 1.0, 0.0)
        v = lax.dot_general(oh, en, (((1,), (0,)), ((), ())),
                            preferred_element_type=jnp.float32,
                            precision=lax.Precision.HIGHEST)
        vecs_ref[0, :, lev, :] = v

    routs_ref[0] = jnp.concatenate(rout_cols, axis=1)

    blk_max = jnp.concatenate(lev_max, axis=0)
    blk_any = jnp.concatenate(lev_any, axis=0)
    stats_ref[m, 0, 0:DEPTH, :] = jnp.maximum(stats_ref[m, 0, 0:DEPTH, :], blk_max)
    stats_ref[m, 1, 0:DEPTH, :] = jnp.maximum(stats_ref[m, 1, 0:DEPTH, :], blk_any)
    for lev in range(DEPTH):
        ec_ref[m, lev] = ec_ref[m, lev] + ec_adds[lev]

    @pl.when((m == NM - 1) & (b == NBLK - 1))
    def _finalize():
        lane1 = lax.broadcasted_iota(jnp.int32, (1, KPAD), 1)
        total = jnp.zeros((1, 1), jnp.float32)
        for mm in range(NM):
            cemax = stats_ref[mm, 0]
            ceany = stats_ref[mm, 1]
            s0 = stats_ref[mm, 2]
            for lev in range(DEPTH):
                off = OFFS[lev]
                k = KS[lev]
                cos = jnp.where(ceany[lev:lev + 1] > 0.5,
                                cemax[lev:lev + 1], s0[lev:lev + 1])
                msk = (lane1 >= off) & (lane1 < off + k)
                ce = 2.0 * (1.0 - jnp.sum(jnp.where(msk, cos, 0.0)) / k)
                ec = 2.0 * (1.0 - ec_ref[mm, lev] / BATCH)
                total = total + ce + ec
        loss_ref[...] = total / (NM * DEPTH)


def kernel(latents_in, emb_weights):
    embcat = jnp.concatenate(emb_weights, axis=0)
    embcat = jnp.pad(embcat, ((0, KPAD - KTOT), (0, 0)))

    en = pl.pallas_call(
        _norm_kernel,
        out_shape=jax.ShapeDtypeStruct((KPAD, DIM), jnp.float32),
    )(embcat)

    routs, vecs, stats, ec, loss = pl.pallas_call(
        _main_kernel,
        grid=(NM, NBLK),
        in_specs=[
            pl.BlockSpec((1, RB, DIM), lambda m, b: (m, b, 0)),
            pl.BlockSpec((KPAD, DIM), lambda m, b: (0, 0)),
        ],
        out_specs=[
            pl.BlockSpec((1, RB, DEPTH), lambda m, b: (m, b, 0)),
            pl.BlockSpec((1, RB, DEPTH, DIM), lambda m, b: (m, b, 0, 0)),
            pl.BlockSpec((NM, 3, DPAD, KPAD), lambda m, b: (0, 0, 0, 0)),
            pl.BlockSpec(memory_space=pltpu.SMEM),
            pl.BlockSpec((1, 1), lambda m, b: (0, 0)),
        ],
        out_shape=[
            jax.ShapeDtypeStruct((NM, BATCH, DEPTH), jnp.int32),
            jax.ShapeDtypeStruct((NM, BATCH, DEPTH, DIM), jnp.float32),
            jax.ShapeDtypeStruct((NM, 3, DPAD, KPAD), jnp.float32),
            jax.ShapeDtypeStruct((NM, DPAD), jnp.float32),
            jax.ShapeDtypeStruct((1, 1), jnp.float32),
        ],
    )(latents_in, en)
    del stats, ec
    return routs, vecs, loss[0, 0]


# speedup vs baseline: 5.4834x; 1.2064x over previous
import jax
import jax.numpy as jnp
from jax import lax
from jax.experimental import pallas as pl
from jax.experimental.pallas import tpu as pltpu

NM = 6
DEPTH = 6
DIM = 300
BATCH = 8192
KS = [2 ** (i + 1) for i in range(DEPTH)]
OFFS = [2 ** (i + 1) - 2 for i in range(DEPTH)]
KTOT = sum(KS)
KPAD = 128
RB = 512
NBLK = BATCH // RB
DPAD = 8


def _norm_kernel(raw_ref, en_ref):
    x = raw_ref[...]
    n = jnp.sqrt(jnp.sum(x * x, axis=1, keepdims=True))
    en_ref[...] = x / jnp.maximum(n, 1e-12)


def _main_kernel(x_ref, en_ref, routs_ref, vecs_ref, stats_ref, ec_ref, loss_ref):
    m = pl.program_id(0)
    b = pl.program_id(1)
    x = x_ref[0]
    n = jnp.sqrt(jnp.sum(x * x, axis=1, keepdims=True))
    xn = x / jnp.maximum(n, 1e-12)
    en = en_ref[...]
    s = lax.dot_general(xn, en, (((1,), (1,)), ((), ())),
                        preferred_element_type=jnp.float32,
                        precision=lax.Precision.DEFAULT)
    d = 1.0 - s
    lane = lax.broadcasted_iota(jnp.int32, (RB, KPAD), 1)

    @pl.when(b == 0)
    def _init():
        stats_ref[m, 0] = jnp.full((DPAD, KPAD), -jnp.inf, jnp.float32)
        stats_ref[m, 1] = jnp.zeros((DPAD, KPAD), jnp.float32)
        stats_ref[m, 2] = jnp.broadcast_to(s[0:1, :], (DPAD, KPAD))
        for lev in range(DEPTH):
            ec_ref[m, lev] = 0.0

    prev = None
    rout_cols = []
    lev_max = []
    lev_any = []
    ec_adds = []
    for lev in range(DEPTH):
        off = OFFS[lev]
        k = KS[lev]
        in_lev = (lane >= off) & (lane < off + k)
        if lev == 0:
            valid = in_lev
        else:
            valid = in_lev & (((lane - off) >> 1) == prev)
        dm = jnp.where(valid, d, jnp.inf)
        dmin = jnp.min(dm, axis=1, keepdims=True)
        hit = valid & (d == dmin)
        gcol = jnp.min(jnp.where(hit, lane, 2 * KPAD), axis=1, keepdims=True)
        prev = gcol - off
        rout_cols.append(prev)
        ec_adds.append(jnp.sum(1.0 - dmin))
        sm = jnp.where(valid, s, -jnp.inf)
        lev_max.append(jnp.max(sm, axis=0, keepdims=True))
        lev_any.append(jnp.max(jnp.where(valid, 1.0, 0.0), axis=0, keepdims=True))
        oh = jnp.where(gcol == lane, 1.0, 0.0)
        v = lax.dot_general(oh, en, (((1,), (0,)), ((), ())),
                            preferred_element_type=jnp.float32,
                            precision=lax.Precision.DEFAULT)
        vecs_ref[0, :, lev, :] = v

    routs_ref[0] = jnp.concatenate(rout_cols, axis=1)

    blk_max = jnp.concatenate(lev_max, axis=0)
    blk_any = jnp.concatenate(lev_any, axis=0)
    stats_ref[m, 0, 0:DEPTH, :] = jnp.maximum(stats_ref[m, 0, 0:DEPTH, :], blk_max)
    stats_ref[m, 1, 0:DEPTH, :] = jnp.maximum(stats_ref[m, 1, 0:DEPTH, :], blk_any)
    for lev in range(DEPTH):
        ec_ref[m, lev] = ec_ref[m, lev] + ec_adds[lev]

    @pl.when((m == NM - 1) & (b == NBLK - 1))
    def _finalize():
        lane1 = lax.broadcasted_iota(jnp.int32, (1, KPAD), 1)
        total = jnp.zeros((1, 1), jnp.float32)
        for mm in range(NM):
            cemax = stats_ref[mm, 0]
            ceany = stats_ref[mm, 1]
            s0 = stats_ref[mm, 2]
            for lev in range(DEPTH):
                off = OFFS[lev]
                k = KS[lev]
                cos = jnp.where(ceany[lev:lev + 1] > 0.5,
                                cemax[lev:lev + 1], s0[lev:lev + 1])
                msk = (lane1 >= off) & (lane1 < off + k)
                ce = 2.0 * (1.0 - jnp.sum(jnp.where(msk, cos, 0.0)) / k)
                ec = 2.0 * (1.0 - ec_ref[mm, lev] / BATCH)
                total = total + ce + ec
        loss_ref[...] = total / (NM * DEPTH)


def kernel(latents_in, emb_weights):
    embcat = jnp.concatenate(emb_weights, axis=0)
    embcat = jnp.pad(embcat, ((0, KPAD - KTOT), (0, 0)))

    en = pl.pallas_call(
        _norm_kernel,
        out_shape=jax.ShapeDtypeStruct((KPAD, DIM), jnp.float32),
    )(embcat)

    routs, vecs, stats, ec, loss = pl.pallas_call(
        _main_kernel,
        grid=(NM, NBLK),
        in_specs=[
            pl.BlockSpec((1, RB, DIM), lambda m, b: (m, b, 0)),
            pl.BlockSpec((KPAD, DIM), lambda m, b: (0, 0)),
        ],
        out_specs=[
            pl.BlockSpec((1, RB, DEPTH), lambda m, b: (m, b, 0)),
            pl.BlockSpec((1, RB, DEPTH, DIM), lambda m, b: (m, b, 0, 0)),
            pl.BlockSpec((NM, 3, DPAD, KPAD), lambda m, b: (0, 0, 0, 0)),
            pl.BlockSpec(memory_space=pltpu.SMEM),
            pl.BlockSpec((1, 1), lambda m, b: (0, 0)),
        ],
        out_shape=[
            jax.ShapeDtypeStruct((NM, BATCH, DEPTH), jnp.int32),
            jax.ShapeDtypeStruct((NM, BATCH, DEPTH, DIM), jnp.float32),
            jax.ShapeDtypeStruct((NM, 3, DPAD, KPAD), jnp.float32),
            jax.ShapeDtypeStruct((NM, DPAD), jnp.float32),
            jax.ShapeDtypeStruct((1, 1), jnp.float32),
        ],
    )(latents_in, en)
    del stats, ec
    return routs, vecs, loss[0, 0]
